# baseline (device time: 30399 ns/iter reference)
import jax
import jax.numpy as jnp
from jax import lax
from jax.experimental import pallas as pl
from jax.experimental.pallas import tpu as pltpu


def kernel(x, router, W1, W2):
    t_half, d = x.shape
    e_loc = W1.shape[0]
    t_full = 2 * t_half

    def body(x_ref, r_ref, w1_ref, w2_ref, out_ref,
             xo_ref, ro_ref, ps_ref, po_ref, send_sems, recv_sems):
        my_x = lax.axis_index("x")
        my_y = lax.axis_index("y")
        my_z = lax.axis_index("z")
        peer = (1 - my_x, my_y, my_z)

        barrier_sem = pltpu.get_barrier_semaphore()
        pl.semaphore_signal(barrier_sem, inc=1, device_id=peer,
                            device_id_type=pl.DeviceIdType.MESH)
        pl.semaphore_wait(barrier_sem, 1)

        rdma_x = pltpu.make_async_remote_copy(
            src_ref=x_ref, dst_ref=xo_ref,
            send_sem=send_sems.at[0], recv_sem=recv_sems.at[0],
            device_id=peer, device_id_type=pl.DeviceIdType.MESH)
        rdma_x.start()
        rdma_r = pltpu.make_async_remote_copy(
            src_ref=r_ref, dst_ref=ro_ref,
            send_sem=send_sems.at[1], recv_sem=recv_sems.at[1],
            device_id=peer, device_id_type=pl.DeviceIdType.MESH)
        rdma_r.start()
        rdma_x.wait()
        rdma_r.wait()

        X = jnp.concatenate([x_ref[:, :], xo_ref[:, :]], axis=0)
        gates = jnp.concatenate(
            [jnp.dot(X, r_ref[:, :], preferred_element_type=jnp.float32),
             jnp.dot(X, ro_ref[:, :], preferred_element_type=jnp.float32)],
            axis=1)

        v1 = jnp.max(gates, axis=1, keepdims=True)
        is1 = gates == v1
        masked = jnp.where(is1, -jnp.inf, gates)
        v2 = jnp.max(masked, axis=1, keepdims=True)
        is2 = masked == v2
        w_top1 = 1.0 / (1.0 + jnp.exp(v2 - v1))
        w_all = jnp.where(is1, w_top1, 0.0) + jnp.where(is2, 1.0 - w_top1, 0.0)

        Xb = X.astype(jnp.bfloat16)
        acc = jnp.zeros((t_full, d), jnp.float32)
        for j in range(e_loc):
            h = jnp.dot(Xb, w1_ref[j].astype(jnp.bfloat16),
                        preferred_element_type=jnp.float32)
            hb = jnp.maximum(h, 0.0).astype(jnp.bfloat16)
            o = jnp.dot(hb, w2_ref[j].astype(jnp.bfloat16),
                        preferred_element_type=jnp.float32)
            acc = acc + o * w_all[:, j:j + 1]

        ps_ref[:, :] = acc[t_half:, :]
        rdma_p = pltpu.make_async_remote_copy(
            src_ref=ps_ref, dst_ref=po_ref,
            send_sem=send_sems.at[2], recv_sem=recv_sems.at[2],
            device_id=peer, device_id_type=pl.DeviceIdType.MESH)
        rdma_p.start()
        rdma_p.wait()
        out_ref[:, :] = acc[:t_half, :] + po_ref[:, :]

    return pl.pallas_call(
        body,
        out_shape=jax.ShapeDtypeStruct((t_half, d), jnp.float32),
        in_specs=[pl.BlockSpec(memory_space=pltpu.VMEM)] * 4,
        out_specs=pl.BlockSpec(memory_space=pltpu.VMEM),
        scratch_shapes=[
            pltpu.VMEM((t_half, d), jnp.float32),
            pltpu.VMEM(router.shape, jnp.float32),
            pltpu.VMEM((t_half, d), jnp.float32),
            pltpu.VMEM((t_half, d), jnp.float32),
            pltpu.SemaphoreType.DMA((3,)),
            pltpu.SemaphoreType.DMA((3,)),
        ],
        compiler_params=pltpu.CompilerParams(collective_id=0),
    )(x, router, W1, W2)


# device time: 24001 ns/iter; 1.2666x vs baseline; 1.2666x over previous
import jax
import jax.numpy as jnp
from jax import lax
from jax.experimental import pallas as pl
from jax.experimental.pallas import tpu as pltpu


def kernel(x, router, W1, W2):
    t_half, d = x.shape
    e_loc = W1.shape[0]

    def body(x_ref, r_ref, w1_ref, w2_ref, out_ref,
             xs_ref, xo_ref, ro_ref, ps_ref, po_ref, send_sems, recv_sems):
        my_x = lax.axis_index("x")
        my_y = lax.axis_index("y")
        my_z = lax.axis_index("z")
        peer = (1 - my_x, my_y, my_z)

        barrier_sem = pltpu.get_barrier_semaphore()
        pl.semaphore_signal(barrier_sem, inc=1, device_id=peer,
                            device_id_type=pl.DeviceIdType.MESH)
        pl.semaphore_wait(barrier_sem, 1)

        xs_ref[:, :] = x_ref[:, :].astype(jnp.bfloat16)
        rdma_x = pltpu.make_async_remote_copy(
            src_ref=xs_ref, dst_ref=xo_ref,
            send_sem=send_sems.at[0], recv_sem=recv_sems.at[0],
            device_id=peer, device_id_type=pl.DeviceIdType.MESH)
        rdma_x.start()
        rdma_r = pltpu.make_async_remote_copy(
            src_ref=r_ref, dst_ref=ro_ref,
            send_sem=send_sems.at[1], recv_sem=recv_sems.at[1],
            device_id=peer, device_id_type=pl.DeviceIdType.MESH)
        rdma_r.start()

        w1b = [w1_ref[j].astype(jnp.bfloat16) for j in range(e_loc)]
        w2b = [w2_ref[j].astype(jnp.bfloat16) for j in range(e_loc)]

        def expert_outs(xb):
            outs = []
            for j in range(e_loc):
                h = jnp.dot(xb, w1b[j], preferred_element_type=jnp.float32)
                hb = jnp.maximum(h, 0.0).astype(jnp.bfloat16)
                outs.append(jnp.dot(hb, w2b[j],
                                    preferred_element_type=jnp.float32))
            return outs

        def local_expert_weights(xb, rm, ro):
            xf = xb.astype(jnp.float32)
            gates = jnp.concatenate(
                [jnp.dot(xf, rm, preferred_element_type=jnp.float32),
                 jnp.dot(xf, ro, preferred_element_type=jnp.float32)],
                axis=1)
            v1 = jnp.max(gates, axis=1, keepdims=True)
            is1 = gates == v1
            masked = jnp.where(is1, -jnp.inf, gates)
            v2 = jnp.max(masked, axis=1, keepdims=True)
            is2 = masked == v2
            w_top1 = 1.0 / (1.0 + jnp.exp(v2 - v1))
            w = (jnp.where(is1, w_top1, 0.0)
                 + jnp.where(is2, 1.0 - w_top1, 0.0))
            return w[:, :e_loc]

        xmb = xs_ref[:, :]
        o_mine = expert_outs(xmb)

        rdma_r.wait()
        rm = r_ref[:, :]
        ro = ro_ref[:, :]
        w_mine = local_expert_weights(xmb, rm, ro)
        acc_mine = sum(o_mine[j] * w_mine[:, j:j + 1] for j in range(e_loc))

        rdma_x.wait()
        xob = xo_ref[:, :]
        w_peer = local_expert_weights(xob, rm, ro)
        o_peer = expert_outs(xob)
        acc_peer = sum(o_peer[j] * w_peer[:, j:j + 1] for j in range(e_loc))

        ps_ref[:, :] = acc_peer.astype(jnp.bfloat16)
        rdma_p = pltpu.make_async_remote_copy(
            src_ref=ps_ref, dst_ref=po_ref,
            send_sem=send_sems.at[2], recv_sem=recv_sems.at[2],
            device_id=peer, device_id_type=pl.DeviceIdType.MESH)
        rdma_p.start()
        rdma_p.wait()
        out_ref[:, :] = acc_mine + po_ref[:, :].astype(jnp.float32)

    return pl.pallas_call(
        body,
        out_shape=jax.ShapeDtypeStruct((t_half, d), jnp.float32),
        in_specs=[pl.BlockSpec(memory_space=pltpu.VMEM)] * 4,
        out_specs=pl.BlockSpec(memory_space=pltpu.VMEM),
        scratch_shapes=[
            pltpu.VMEM((t_half, d), jnp.bfloat16),
            pltpu.VMEM((t_half, d), jnp.bfloat16),
            pltpu.VMEM(router.shape, jnp.float32),
            pltpu.VMEM((t_half, d), jnp.bfloat16),
            pltpu.VMEM((t_half, d), jnp.bfloat16),
            pltpu.SemaphoreType.DMA((3,)),
            pltpu.SemaphoreType.DMA((3,)),
        ],
        compiler_params=pltpu.CompilerParams(collective_id=0),
    )(x, router, W1, W2)


# device time: 18629 ns/iter; 1.6318x vs baseline; 1.2884x over previous
import jax
import jax.numpy as jnp
from jax import lax
from jax.experimental import pallas as pl
from jax.experimental.pallas import tpu as pltpu

N_CHUNKS = 2


def kernel(x, router, W1, W2):
    t_half, d = x.shape
    e_loc = W1.shape[0]
    tc = t_half // N_CHUNKS

    def body(x_ref, rt_ref, w1_ref, w2_ref, out_ref,
             xo_ref, rv_ref, ps_ref, po_ref, w1c_ref, w2c_ref,
             send_sems, recv_sems):
        my_x = lax.axis_index("x")
        my_y = lax.axis_index("y")
        my_z = lax.axis_index("z")
        peer = (1 - my_x, my_y, my_z)

        barrier_sem = pltpu.get_barrier_semaphore()
        pl.semaphore_signal(barrier_sem, inc=1, device_id=peer,
                            device_id_type=pl.DeviceIdType.MESH)
        pl.semaphore_wait(barrier_sem, 1)

        def chunk_rdma(src, dst, sem_i):
            return pltpu.make_async_remote_copy(
                src_ref=src, dst_ref=dst,
                send_sem=send_sems.at[sem_i], recv_sem=recv_sems.at[sem_i],
                device_id=peer, device_id_type=pl.DeviceIdType.MESH)

        rdma_x = [
            chunk_rdma(x_ref.at[pl.ds(c * tc, tc), :],
                       xo_ref.at[pl.ds(c * tc, tc), :], c)
            for c in range(N_CHUNKS)
        ]
        for r in rdma_x:
            r.start()
        rdma_r = chunk_rdma(rt_ref, rv_ref, N_CHUNKS)
        rdma_r.start()

        for j in range(e_loc):
            w1c_ref[j, :, :] = w1_ref[j, :, :].astype(jnp.bfloat16)
            w2c_ref[j, :, :] = w2_ref[j, :, :].astype(jnp.bfloat16)

        def expert_outs(xb):
            outs = []
            for j in range(e_loc):
                h = jnp.dot(xb, w1c_ref[j, :, :],
                            preferred_element_type=jnp.float32)
                hb = jnp.maximum(h, 0.0).astype(jnp.bfloat16)
                outs.append(jnp.dot(hb, w2c_ref[j, :, :],
                                    preferred_element_type=jnp.float32))
            return outs

        def local_expert_weights(xb, rm_t, ro_t):
            xf = xb.astype(jnp.float32)
            g_m = lax.dot_general(
                xf, rm_t, (((1,), (1,)), ((), ())),
                preferred_element_type=jnp.float32)
            g_o = lax.dot_general(
                xf, ro_t, (((1,), (1,)), ((), ())),
                preferred_element_type=jnp.float32)
            gates = jnp.concatenate([g_m, g_o], axis=1)
            v1 = jnp.max(gates, axis=1, keepdims=True)
            is1 = gates == v1
            masked = jnp.where(is1, -jnp.inf, gates)
            v2 = jnp.max(masked, axis=1, keepdims=True)
            is2 = masked == v2
            w_top1 = 1.0 / (1.0 + jnp.exp(v2 - v1))
            w = (jnp.where(is1, w_top1, 0.0)
                 + jnp.where(is2, 1.0 - w_top1, 0.0))
            return w[:, :e_loc]

        def weighted(xb, rm_t, ro_t):
            w = local_expert_weights(xb, rm_t, ro_t)
            o = expert_outs(xb)
            return sum(o[j] * w[:, j:j + 1] for j in range(e_loc))

        xmb = x_ref[:, :]
        o_mine = expert_outs(xmb)

        rdma_r.wait()
        rm_t = rt_ref[:, :]
        ro_t = rv_ref[:, :]
        w_mine = local_expert_weights(xmb, rm_t, ro_t)
        acc_mine = sum(o_mine[j] * w_mine[:, j:j + 1] for j in range(e_loc))

        rdma_p = [
            chunk_rdma(ps_ref.at[pl.ds(c * tc, tc), :],
                       po_ref.at[pl.ds(c * tc, tc), :], N_CHUNKS + 1 + c)
            for c in range(N_CHUNKS)
        ]
        for c in range(N_CHUNKS):
            rdma_x[c].wait()
            sl = pl.ds(c * tc, tc)
            ps_ref[sl, :] = weighted(
                xo_ref[sl, :], rm_t, ro_t).astype(jnp.bfloat16)
            rdma_p[c].start()

        for c in range(N_CHUNKS):
            rdma_p[c].wait()
            sl = pl.ds(c * tc, tc)
            out_ref[sl, :] = (acc_mine[c * tc:(c + 1) * tc, :]
                              + po_ref[sl, :].astype(jnp.float32))

    x_b = x.astype(jnp.bfloat16)
    router_t = jnp.swapaxes(router, 0, 1)
    n_sems = 2 * N_CHUNKS + 1

    return pl.pallas_call(
        body,
        out_shape=jax.ShapeDtypeStruct((t_half, d), jnp.float32),
        in_specs=[pl.BlockSpec(memory_space=pltpu.VMEM)] * 4,
        out_specs=pl.BlockSpec(memory_space=pltpu.VMEM),
        scratch_shapes=[
            pltpu.VMEM((t_half, d), jnp.bfloat16),
            pltpu.VMEM((e_loc, d), jnp.float32),
            pltpu.VMEM((t_half, d), jnp.bfloat16),
            pltpu.VMEM((t_half, d), jnp.bfloat16),
            pltpu.VMEM(W1.shape, jnp.bfloat16),
            pltpu.VMEM(W2.shape, jnp.bfloat16),
            pltpu.SemaphoreType.DMA((n_sems,)),
            pltpu.SemaphoreType.DMA((n_sems,)),
        ],
        compiler_params=pltpu.CompilerParams(collective_id=0),
    )(x_b, router_t, W1, W2)
